# Initial kernel scaffold; baseline (speedup 1.0000x reference)
import jax
import jax.numpy as jnp
from jax.experimental import pallas as pl


def kernel(x, Wc, b):
    raise NotImplementedError("write your pallas kernel here")


# speedup vs baseline: 18.8814x; 18.8814x over previous
import functools

import jax
import jax.numpy as jnp
from jax import lax
from jax.experimental import pallas as pl
from jax.experimental.pallas import tpu as pltpu
from jax.experimental.pallas import tpu_sc as plsc

_B, _C, _N, _K, _OUT = 2, 96, 4096, 9, 96
_ROWS = 256
_KPAD = 16
_CHUNK = 64
_NBIG = 3.0e38


def _topk_body(x_ref, idx_ref):
    b = pl.program_id(0)
    j = pl.program_id(1)

    xall = x_ref[0]
    nrm = jnp.sqrt(jnp.sum(xall * xall, axis=1, keepdims=True))
    xn = xall / jnp.maximum(nrm, 1e-12)
    xsq = jnp.sum(xn * xn, axis=1, keepdims=True)

    rraw = x_ref[0, pl.ds(j * _ROWS, _ROWS), :]
    rnrm = jnp.sqrt(jnp.sum(rraw * rraw, axis=1, keepdims=True))
    rows = rraw / jnp.maximum(rnrm, 1e-12)
    rsq = jnp.sum(rows * rows, axis=1, keepdims=True)
    rows_aug = jnp.concatenate(
        [rows, jnp.ones((_ROWS, 1), jnp.float32)], axis=1)
    cols_aug = jnp.concatenate([-2.0 * xn, xsq], axis=1)
    d = lax.dot_general(rows_aug, cols_aug, (((1,), (1,)), ((), ())),
                        preferred_element_type=jnp.float32)
    d = d + rsq

    lane = lax.broadcasted_iota(jnp.int32, (1, _N), 1)
    bias = b * _N
    idx_ref[...] = jnp.zeros((1, _KPAD, _ROWS), jnp.int32)
    for k in range(_K):
        m = jnp.min(d, axis=1, keepdims=True)
        cand = jnp.where(d == m, lane, _N)
        sel = jnp.min(cand, axis=1)
        idx_ref[0, k, :] = sel + bias
        d = jnp.where(lane == sel[:, None], _NBIG, d)


def _build_topk():
    grid = (_B, _N // _ROWS)
    return pl.pallas_call(
        _topk_body,
        grid=grid,
        in_specs=[pl.BlockSpec((1, _N, _C), lambda b, j: (b, 0, 0))],
        out_specs=pl.BlockSpec((1, _KPAD, _ROWS), lambda b, j: (b, 0, j)),
        out_shape=jax.ShapeDtypeStruct((_B, _KPAD, _N), jnp.int32),
    )


_CP = 128


def _sc_info():
    info = plsc.get_sparse_core_info()
    return info.num_cores, info.num_subcores


def _build_gather_max():
    nc, ns = _sc_info()
    nw = nc * ns
    per_w = (_B * _N) // nw
    nchunks = per_w // _CHUNK
    mesh = plsc.VectorSubcoreMesh(core_axis_name="c", subcore_axis_name="s")

    @functools.partial(
        pl.kernel,
        mesh=mesh,
        out_type=jax.ShapeDtypeStruct((_B * _N, _CP), jnp.float32),
        scratch_types=[
            pltpu.VMEM((_K, _CHUNK), jnp.int32),
            pltpu.VMEM((_K * _CHUNK, _CP), jnp.float32),
            pltpu.VMEM((_CHUNK, _CP), jnp.float32),
            pltpu.VMEM((_CHUNK, _CP), jnp.float32),
            pltpu.SemaphoreType.DMA,
        ],
    )
    def gather_max(table_hbm, idx_hbm, out_hbm, idx_v, rows_v, cen_v,
                   res_v, sem):
        wid = lax.axis_index("s") * nc + lax.axis_index("c")

        def chunk_body(ch):
            g0 = wid * per_w + ch * _CHUNK
            bb = g0 // _N
            n0 = g0 - bb * _N
            for k in range(_K):
                off = bb * (_KPAD * _N) + k * _N + n0
                pltpu.sync_copy(idx_hbm.at[pl.ds(off, _CHUNK)],
                                idx_v.at[k])
            copies = [
                pltpu.async_copy(table_hbm.at[idx_v.at[k]],
                                 rows_v.at[pl.ds(k * _CHUNK, _CHUNK)], sem)
                for k in range(_K)
            ]
            pltpu.sync_copy(table_hbm.at[pl.ds(g0, _CHUNK)], cen_v)
            for c in copies:
                c.wait()

            def node_body(dn):
                for cb in range(_C // 16):
                    sl = pl.ds(cb * 16, 16)
                    acc = rows_v[dn, sl]
                    for k in range(1, _K):
                        acc = jnp.maximum(acc, rows_v[k * _CHUNK + dn, sl])
                    res_v[dn, sl] = acc - cen_v[dn, sl]

            pl.loop(0, _CHUNK)(node_body)
            pltpu.sync_copy(res_v, out_hbm.at[pl.ds(g0, _CHUNK)])

        pl.loop(0, nchunks)(chunk_body)

    return gather_max


def _conv_body(xf_ref, xm_ref, we_ref, wo_ref, b_ref, out_ref):
    a = lax.dot_general(we_ref[...], xf_ref[0], (((1,), (1,)), ((), ())),
                        preferred_element_type=jnp.float32)
    m = lax.dot_general(wo_ref[...], xm_ref[0][:, :_C],
                        (((1,), (1,)), ((), ())),
                        preferred_element_type=jnp.float32)
    out_ref[0] = jnp.maximum(a + m + b_ref[...], 0.0)


def _build_conv():
    cols = 512
    grid = (_B, _N // cols)
    return pl.pallas_call(
        _conv_body,
        grid=grid,
        in_specs=[
            pl.BlockSpec((1, cols, _C), lambda b, j: (b, j, 0)),
            pl.BlockSpec((1, cols, _CP), lambda b, j: (b, j, 0)),
            pl.BlockSpec((_OUT, _C), lambda b, j: (0, 0)),
            pl.BlockSpec((_OUT, _C), lambda b, j: (0, 0)),
            pl.BlockSpec((_OUT, 1), lambda b, j: (0, 0)),
        ],
        out_specs=pl.BlockSpec((1, _OUT, cols), lambda b, j: (b, 0, j)),
        out_shape=jax.ShapeDtypeStruct((_B, _OUT, _N), jnp.float32),
    )


def kernel(x, Wc, b):
    Bs, Cs, Hs, Ws, Ds = x.shape
    N = Hs * Ws * Ds
    xf = x.reshape(Bs, Cs, N)
    xfT = jnp.transpose(xf, (0, 2, 1))

    nn_idx = _build_topk()(xfT)

    table = jnp.pad(xfT.reshape(Bs * N, Cs), ((0, 0), (0, _CP - Cs)))
    xjmax = _build_gather_max()(table, nn_idx.reshape(-1))
    xjmax = xjmax.reshape(Bs, N, _CP)

    we = Wc[:, 0::2]
    wo = Wc[:, 1::2]
    out = _build_conv()(xfT, xjmax, we, wo, b.reshape(_OUT, 1))
    return out.reshape(Bs, _OUT, Hs, Ws, Ds)


# speedup vs baseline: 23.6746x; 1.2539x over previous
import functools

import jax
import jax.numpy as jnp
from jax import lax
from jax.experimental import pallas as pl
from jax.experimental.pallas import tpu as pltpu
from jax.experimental.pallas import tpu_sc as plsc

_B, _C, _N, _K, _OUT = 2, 96, 4096, 9, 96
_ROWS = 256
_KPAD = 16
_CHUNK = 64
_CP = 128
_NBIG = 3.0e38


def _topk_body(x_ref, idx_ref):
    b = pl.program_id(0)
    j = pl.program_id(1)

    xall = x_ref[0]
    nrm = jnp.sqrt(jnp.sum(xall * xall, axis=1, keepdims=True))
    xn = xall / jnp.maximum(nrm, 1e-12)
    xsq = jnp.sum(xn * xn, axis=1, keepdims=True)

    rraw = x_ref[0, pl.ds(j * _ROWS, _ROWS), :]
    rnrm = jnp.sqrt(jnp.sum(rraw * rraw, axis=1, keepdims=True))
    rows = rraw / jnp.maximum(rnrm, 1e-12)
    rsq = jnp.sum(rows * rows, axis=1, keepdims=True)
    rows_aug = jnp.concatenate(
        [rows, jnp.ones((_ROWS, 1), jnp.float32)], axis=1)
    cols_aug = jnp.concatenate([-2.0 * xn, xsq], axis=1)
    d = lax.dot_general(rows_aug, cols_aug, (((1,), (1,)), ((), ())),
                        preferred_element_type=jnp.float32)
    d = d + rsq

    lane = lax.broadcasted_iota(jnp.int32, (1, _N), 1).astype(jnp.float32)
    lane128 = lax.broadcasted_iota(jnp.int32, (1, 128), 1)
    bias = b * _N
    acc = jnp.zeros((_ROWS, 128), jnp.float32)
    for k in range(_K):
        m = jnp.min(d, axis=1, keepdims=True)
        cand = jnp.where(d == m, lane, _NBIG)
        sel = jnp.min(cand, axis=1, keepdims=True)
        acc = jnp.where(lane128 == k, sel, acc)
        d = jnp.where(lane == sel, _NBIG, d)
    accT = jnp.transpose(acc, (1, 0))
    idx_ref[0] = accT[:_KPAD, :].astype(jnp.int32) + bias


def _build_topk():
    grid = (_B, _N // _ROWS)
    return pl.pallas_call(
        _topk_body,
        grid=grid,
        in_specs=[pl.BlockSpec((1, _N, _CP), lambda b, j: (b, 0, 0))],
        out_specs=pl.BlockSpec((1, _KPAD, _ROWS), lambda b, j: (b, 0, j)),
        out_shape=jax.ShapeDtypeStruct((_B, _KPAD, _N), jnp.int32),
    )


def _sc_info():
    info = plsc.get_sparse_core_info()
    return info.num_cores, info.num_subcores


def _build_gather_max():
    nc, ns = _sc_info()
    nw = nc * ns
    per_w = (_B * _N) // nw
    nchunks = per_w // _CHUNK
    mesh = plsc.VectorSubcoreMesh(core_axis_name="c", subcore_axis_name="s")

    @functools.partial(
        pl.kernel,
        mesh=mesh,
        out_type=jax.ShapeDtypeStruct((_B * _N, _CP), jnp.float32),
        scratch_types=[
            pltpu.VMEM((_K, _CHUNK), jnp.int32),
            pltpu.VMEM((_K * _CHUNK, _CP), jnp.float32),
            pltpu.VMEM((_CHUNK, _CP), jnp.float32),
            pltpu.VMEM((_CHUNK, _CP), jnp.float32),
            pltpu.SemaphoreType.DMA,
        ],
    )
    def gather_max(table_hbm, idx_hbm, out_hbm, idx_v, rows_v,
                   cen_v, res_v, sem):
        wid = lax.axis_index("s") * nc + lax.axis_index("c")

        def chunk_body(ch):
            g0 = wid * per_w + ch * _CHUNK
            bb = g0 // _N
            n0 = g0 - bb * _N
            for k in range(_K):
                off = bb * (_KPAD * _N) + k * _N + n0
                pltpu.sync_copy(idx_hbm.at[pl.ds(off, _CHUNK)],
                                idx_v.at[k])
            copies = [
                pltpu.async_copy(table_hbm.at[idx_v.at[k]],
                                 rows_v.at[pl.ds(k * _CHUNK, _CHUNK)], sem)
                for k in range(_K)
            ]
            pltpu.sync_copy(table_hbm.at[pl.ds(g0, _CHUNK)], cen_v)
            for c in copies:
                c.wait()

            def node_body(dn):
                for cb in range(_C // 16):
                    sl = pl.ds(cb * 16, 16)
                    acc = rows_v[dn, sl]
                    for k in range(1, _K):
                        acc = jnp.maximum(acc, rows_v[k * _CHUNK + dn, sl])
                    res_v[dn, sl] = acc - cen_v[dn, sl]

            pl.loop(0, _CHUNK)(node_body)
            pltpu.sync_copy(res_v, out_hbm.at[pl.ds(g0, _CHUNK)])

        pl.loop(0, nchunks)(chunk_body)

    return gather_max


def _conv_body(xf_ref, xm_ref, we_ref, wo_ref, b_ref, out_ref):
    a = lax.dot_general(we_ref[...], xf_ref[0][:, :_C],
                        (((1,), (1,)), ((), ())),
                        preferred_element_type=jnp.float32)
    m = lax.dot_general(wo_ref[...], xm_ref[0][:, :_C],
                        (((1,), (1,)), ((), ())),
                        preferred_element_type=jnp.float32)
    out_ref[0] = jnp.maximum(a + m + b_ref[...], 0.0)


def _build_conv():
    cols = 512
    grid = (_B, _N // cols)
    return pl.pallas_call(
        _conv_body,
        grid=grid,
        in_specs=[
            pl.BlockSpec((1, cols, _CP), lambda b, j: (b, j, 0)),
            pl.BlockSpec((1, cols, _CP), lambda b, j: (b, j, 0)),
            pl.BlockSpec((_OUT, _C), lambda b, j: (0, 0)),
            pl.BlockSpec((_OUT, _C), lambda b, j: (0, 0)),
            pl.BlockSpec((_OUT, 1), lambda b, j: (0, 0)),
        ],
        out_specs=pl.BlockSpec((1, _OUT, cols), lambda b, j: (b, 0, j)),
        out_shape=jax.ShapeDtypeStruct((_B, _OUT, _N), jnp.float32),
    )


def kernel(x, Wc, b):
    Bs, Cs, Hs, Ws, Ds = x.shape
    N = Hs * Ws * Ds
    xf = x.reshape(Bs, Cs, N)
    xpad = jnp.pad(jnp.transpose(xf, (0, 2, 1)),
                   ((0, 0), (0, 0), (0, _CP - Cs)))

    nn_idx = _build_topk()(xpad)

    table = xpad.reshape(Bs * N, _CP)
    xjmax = _build_gather_max()(table, nn_idx.reshape(-1))
    xjmax = xjmax.reshape(Bs, N, _CP)

    we = Wc[:, 0::2]
    wo = Wc[:, 1::2]
    out = _build_conv()(xpad, xjmax, we, wo, b.reshape(_OUT, 1))
    return out.reshape(Bs, _OUT, Hs, Ws, Ds)


# speedup vs baseline: 27.7790x; 1.1734x over previous
import functools

import jax
import jax.numpy as jnp
from jax import lax
from jax.experimental import pallas as pl
from jax.experimental.pallas import tpu as pltpu
from jax.experimental.pallas import tpu_sc as plsc

_B, _C, _N, _K, _OUT = 2, 96, 4096, 9, 96
_ROWS = 256
_KPAD = 16
_CHUNK = 64
_CP = 128
_NBIG = 3.0e38


def _topk_body(bias, x_ref, idx_ref, rows_s, cols_s):
    j = pl.program_id(0)

    @pl.when(j == 0)
    def _prep():
        xall = x_ref[...]
        nrm = jnp.sqrt(jnp.sum(xall * xall, axis=1, keepdims=True))
        xn = xall / jnp.maximum(nrm, 1e-12)
        xsq = jnp.sum(xn * xn, axis=1, keepdims=True)
        lane = lax.broadcasted_iota(jnp.int32, (1, _CP), 1)
        cols = -2.0 * xn
        cols = jnp.where(lane == _C, xsq, cols)
        cols = jnp.where(lane == _C + 1, 1.0, cols)
        rows = jnp.where(lane == _C, 1.0, xn)
        rows = jnp.where(lane == _C + 1, xsq, rows)
        rows_s[...] = rows
        cols_s[...] = cols

    @pl.when(j > 0)
    def _block():
        jj = j - 1
        rows = rows_s[pl.ds(jj * _ROWS, _ROWS), :]
        d = lax.dot_general(rows, cols_s[...], (((1,), (1,)), ((), ())),
                            preferred_element_type=jnp.float32)

        lane = lax.broadcasted_iota(
            jnp.int32, (1, _N), 1).astype(jnp.float32)
        lane128 = lax.broadcasted_iota(jnp.int32, (1, 128), 1)
        acc = jnp.zeros((_ROWS, 128), jnp.float32)
        for k in range(_K):
            m = jnp.min(d, axis=1, keepdims=True)
            cand = jnp.where(d == m, lane, _NBIG)
            sel = jnp.min(cand, axis=1, keepdims=True)
            acc = jnp.where(lane128 == k, sel, acc)
            d = jnp.where(lane == sel, _NBIG, d)
        accT = jnp.transpose(acc, (1, 0))
        idx_ref[...] = accT[:_KPAD, :].astype(jnp.int32) + bias


def _build_topk(b):
    return pl.pallas_call(
        functools.partial(_topk_body, b * _N),
        grid=(_N // _ROWS + 1,),
        in_specs=[pl.BlockSpec((_N, _CP), lambda j: (0, 0))],
        out_specs=pl.BlockSpec(
            (_KPAD, _ROWS), lambda j: (0, jnp.maximum(j - 1, 0))),
        out_shape=jax.ShapeDtypeStruct((_KPAD, _N), jnp.int32),
        scratch_shapes=[
            pltpu.VMEM((_N, _CP), jnp.float32),
            pltpu.VMEM((_N, _CP), jnp.float32),
        ],
    )


def _sc_info():
    info = plsc.get_sparse_core_info()
    return info.num_cores, info.num_subcores


def _build_gather_max(b):
    nc, ns = _sc_info()
    nw = nc * ns
    per_w = _N // nw
    nchunks = per_w // _CHUNK
    base_b = b * _N
    mesh = plsc.VectorSubcoreMesh(core_axis_name="c", subcore_axis_name="s")

    @functools.partial(
        pl.kernel,
        mesh=mesh,
        out_type=jax.ShapeDtypeStruct((_N, _CP), jnp.float32),
        scratch_types=[
            pltpu.VMEM((_K, _CHUNK), jnp.int32),
            pltpu.VMEM((_K * _CHUNK, _CP), jnp.float32),
            pltpu.VMEM((_CHUNK, _CP), jnp.float32),
            pltpu.VMEM((_CHUNK, _CP), jnp.float32),
            pltpu.SemaphoreType.DMA,
        ],
    )
    def gather_max(table_hbm, idx_hbm, out_hbm, idx_v, rows_v,
                   cen_v, res_v, sem):
        wid = lax.axis_index("s") * nc + lax.axis_index("c")

        def chunk_body(ch):
            n0 = wid * per_w + ch * _CHUNK
            for k in range(_K):
                pltpu.sync_copy(idx_hbm.at[pl.ds(k * _N + n0, _CHUNK)],
                                idx_v.at[k])
            copies = [
                pltpu.async_copy(table_hbm.at[idx_v.at[k]],
                                 rows_v.at[pl.ds(k * _CHUNK, _CHUNK)], sem)
                for k in range(_K)
            ]
            pltpu.sync_copy(table_hbm.at[pl.ds(base_b + n0, _CHUNK)], cen_v)
            for c in copies:
                c.wait()

            def node_body(dn):
                for cb in range(_C // 16):
                    sl = pl.ds(cb * 16, 16)
                    acc = rows_v[dn, sl]
                    for k in range(1, _K):
                        acc = jnp.maximum(acc, rows_v[k * _CHUNK + dn, sl])
                    res_v[dn, sl] = acc - cen_v[dn, sl]

            pl.loop(0, _CHUNK)(node_body)
            pltpu.sync_copy(res_v, out_hbm.at[pl.ds(n0, _CHUNK)])

        pl.loop(0, nchunks)(chunk_body)

    return gather_max


def _conv_body(xf_ref, xm_ref, we_ref, wo_ref, b_ref, out_ref):
    a = lax.dot_general(we_ref[...], xf_ref[...][:, :_C],
                        (((1,), (1,)), ((), ())),
                        preferred_element_type=jnp.float32)
    m = lax.dot_general(wo_ref[...], xm_ref[...][:, :_C],
                        (((1,), (1,)), ((), ())),
                        preferred_element_type=jnp.float32)
    out_ref[...] = jnp.maximum(a + m + b_ref[...], 0.0)


def _build_conv():
    cols = 512
    return pl.pallas_call(
        _conv_body,
        grid=(_N // cols,),
        in_specs=[
            pl.BlockSpec((cols, _CP), lambda j: (j, 0)),
            pl.BlockSpec((cols, _CP), lambda j: (j, 0)),
            pl.BlockSpec((_OUT, _C), lambda j: (0, 0)),
            pl.BlockSpec((_OUT, _C), lambda j: (0, 0)),
            pl.BlockSpec((_OUT, 1), lambda j: (0, 0)),
        ],
        out_specs=pl.BlockSpec((_OUT, cols), lambda j: (0, j)),
        out_shape=jax.ShapeDtypeStruct((_OUT, _N), jnp.float32),
    )


def kernel(x, Wc, b):
    Bs, Cs, Hs, Ws, Ds = x.shape
    N = Hs * Ws * Ds
    xf = x.reshape(Bs, Cs, N)
    xpad = jnp.pad(jnp.transpose(xf, (0, 2, 1)),
                   ((0, 0), (0, 0), (0, _CP - Cs)))
    table = xpad.reshape(Bs * N, _CP)

    we = Wc[:, 0::2]
    wo = Wc[:, 1::2]
    b2 = b.reshape(_OUT, 1)
    conv = _build_conv()

    outs = []
    for bb in range(Bs):
        nn_idx = _build_topk(bb)(xpad[bb])
        xj = _build_gather_max(bb)(table, nn_idx.reshape(-1))
        outs.append(conv(xpad[bb], xj, we, wo, b2))
    out = jnp.stack(outs)
    return out.reshape(Bs, _OUT, Hs, Ws, Ds)
